# baseline (speedup 1.0000x reference)
import functools

import jax
import jax.numpy as jnp
from jax import lax
from jax.experimental import pallas as pl
from jax.experimental.pallas import tpu as pltpu
from jax.experimental.pallas import tpu_sc as plsc

IDENTITIES = 1000
LENGTH = 320
WEIGHT_DIM = 150
PAD_DIM = 152
BATCH = 1024

NUM_CORES = 2
NUM_SUBCORES = 16
NUM_WORKERS = NUM_CORES * NUM_SUBCORES

L_PER_WORKER = LENGTH // NUM_WORKERS
CHUNK = 128
CHUNKS_PER_L = BATCH // CHUNK
CHUNKS_PER_WORKER = L_PER_WORKER * CHUNKS_PER_L


def _sc_gather(table, idx):
    mesh = plsc.VectorSubcoreMesh(core_axis_name="c", subcore_axis_name="s")

    @functools.partial(
        pl.kernel,
        mesh=mesh,
        out_type=jax.ShapeDtypeStruct((LENGTH * BATCH, PAD_DIM), jnp.float32),
        compiler_params=pltpu.CompilerParams(use_tc_tiling_on_sc=False),
        scratch_types=[
            pltpu.VMEM((BATCH,), jnp.int32),
            pltpu.VMEM((BATCH,), jnp.int32),
            pltpu.VMEM((CHUNK,), jnp.int32),
            pltpu.VMEM((CHUNK, PAD_DIM), jnp.float32),
            pltpu.SemaphoreType.DMA,
        ],
    )
    def k(table_hbm, idx_hbm, out_hbm, idx_v, scaled_v, idxc_v, buf_v, sem):
        wid = lax.axis_index("s") * NUM_CORES + lax.axis_index("c")
        pltpu.sync_copy(idx_hbm, idx_v)

        @pl.loop(0, BATCH // 16)
        def _scale(i):
            s = pl.ds(i * 16, 16)
            scaled_v[s] = idx_v[s] * LENGTH

        l_base = wid * L_PER_WORKER

        @pl.loop(0, CHUNKS_PER_WORKER)
        def _chunk(kk):
            l = l_base + kk // CHUNKS_PER_L
            b0 = (kk % CHUNKS_PER_L) * CHUNK

            @pl.loop(0, CHUNK // 16)
            def _mkidx(i):
                idxc_v[pl.ds(i * 16, 16)] = scaled_v[pl.ds(b0 + i * 16, 16)] + l

            pltpu.async_copy(table_hbm.at[idxc_v], buf_v, sem).wait()
            pltpu.sync_copy(buf_v, out_hbm.at[pl.ds(l * BATCH + b0, CHUNK)])

    return k(table, idx)


def kernel(weights, ref_img):
    table = weights.reshape(IDENTITIES * LENGTH, WEIGHT_DIM)
    table = jnp.pad(table, ((0, 0), (0, PAD_DIM - WEIGHT_DIM)))
    idx = ref_img.astype(jnp.int32)
    out = _sc_gather(table, idx)
    return out[:, :WEIGHT_DIM].reshape(LENGTH, BATCH, WEIGHT_DIM)
